# Initial kernel scaffold; baseline (speedup 1.0000x reference)
import jax
import jax.numpy as jnp
from jax.experimental import pallas as pl


def kernel(x, edge_index, adj_values, W):
    raise NotImplementedError("write your pallas kernel here")


# speedup vs baseline: 2.8489x; 2.8489x over previous
import functools

import jax
import jax.numpy as jnp
from jax import lax
from jax.experimental import pallas as pl
from jax.experimental.pallas import tpu as pltpu
from jax.experimental.pallas import tpu_sc as plsc

_N = 10000
_E = 320000
_D = 128

_NC = 2
_NS = 16
_CH = 128
_CPT = 80
_EPT = _CH * _CPT
_E_PAD = _EPT * _NS * _NC
_NPAD = 10240
_RP = _NPAD // _NS


def _mm_body(x_ref, w_ref, o_ref):
    o_ref[...] = jnp.dot(x_ref[...], w_ref[...],
                         preferred_element_type=jnp.float32)


def _matmul(x, W):
    return pl.pallas_call(
        _mm_body,
        grid=(10,),
        in_specs=[
            pl.BlockSpec((1000, _D), lambda r: (r, 0)),
            pl.BlockSpec((_D, _D), lambda r: (0, 0)),
        ],
        out_specs=pl.BlockSpec((1000, _D), lambda r: (r, 0)),
        out_shape=jax.ShapeDtypeStruct((_N, _D), jnp.float32),
    )(x, W)


def _add_body(a_ref, b_ref, o_ref):
    o_ref[...] = a_ref[0] + b_ref[0]


def _combine(parts):
    return pl.pallas_call(
        _add_body,
        grid=(10,),
        in_specs=[
            pl.BlockSpec((1, 1000, _D), lambda r: (0, r, 0)),
            pl.BlockSpec((1, 1000, _D), lambda r: (1, r, 0)),
        ],
        out_specs=pl.BlockSpec((1000, _D), lambda r: (r, 0)),
        out_shape=jax.ShapeDtypeStruct((_N, _D), jnp.float32),
    )(parts, parts)


@functools.partial(
    pl.kernel,
    out_type=jax.ShapeDtypeStruct((_NC, _NPAD, _D), jnp.float32),
    mesh=plsc.VectorSubcoreMesh(core_axis_name="c", subcore_axis_name="s"),
    scratch_types=[
        pltpu.VMEM((_CH,), jnp.int32),
        pltpu.VMEM((_CH,), jnp.int32),
        pltpu.VMEM((_CH,), jnp.float32),
        pltpu.VMEM((_CH, _D), jnp.float32),
        pltpu.VMEM_SHARED((_NPAD, _D), jnp.float32),
        pltpu.SemaphoreType.DMA,
    ],
)
def _sc_agg(hs_hbm, cols_hbm, rows_hbm, vals_hbm, out_hbm,
            colbuf, rowbuf, valbuf, rbuf, acc, gsem):
    c = lax.axis_index("c")
    s = lax.axis_index("s")

    def _zrow(r, carry):
        for j in range(_D // 16):
            rbuf[r, pl.ds(j * 16, 16)] = jnp.zeros((16,), jnp.float32)
        return carry
    lax.fori_loop(0, _CH, _zrow, 0)
    for k in range(_RP // _CH):
        pltpu.sync_copy(rbuf, acc.at[pl.ds(s * _RP + k * _CH, _CH), :])
    plsc.subcore_barrier()

    tile_base = c * (_E_PAD // _NC) + s * _EPT

    def _chunk(g, carry):
        base = tile_base + g * _CH
        pltpu.sync_copy(cols_hbm.at[pl.ds(base, _CH)], colbuf)
        pltpu.sync_copy(rows_hbm.at[pl.ds(base, _CH)], rowbuf)
        pltpu.sync_copy(vals_hbm.at[pl.ds(base, _CH)], valbuf)
        pltpu.async_copy(hs_hbm.at[colbuf], rbuf, gsem).wait()

        def _edge16(g16, carry2):
            vv = valbuf[pl.ds(g16 * 16, 16)]
            for i in range(16):
                e = g16 * 16 + i
                sp = vv[i]
                for j in range(_D // 16):
                    rbuf[e, pl.ds(j * 16, 16)] = (
                        rbuf[e, pl.ds(j * 16, 16)] * sp)
            return carry2
        lax.fori_loop(0, _CH // 16, _edge16, 0)

        pltpu.sync_copy(rbuf, acc.at[rowbuf], add=True)
        return carry
    lax.fori_loop(0, _CPT, _chunk, 0)

    plsc.subcore_barrier()
    pltpu.sync_copy(acc.at[pl.ds(s * _RP, _RP), :],
                    out_hbm.at[c, pl.ds(s * _RP, _RP), :])


def kernel(x, edge_index, adj_values, W):
    ei = edge_index.astype(jnp.int32)
    pad = _E_PAD - _E
    rows_p = jnp.pad(ei[0], (0, pad))
    cols_p = jnp.pad(ei[1], (0, pad))
    vals_p = jnp.pad(adj_values, (0, pad))
    h = _matmul(x, W)
    parts = _sc_agg(h, cols_p, rows_p, vals_p)
    return _combine(parts)
